# baseline (speedup 1.0000x reference)
import functools

import jax
import jax.numpy as jnp
from jax import lax
from jax.experimental import pallas as pl
from jax.experimental.pallas import tpu as pltpu
from jax.experimental.pallas import tpu_sc as plsc

_NC, _NS = 2, 16


def _chunk_size(n, cap=128):
    best = 0
    for c in range(8, cap + 1, 8):
        if n % c == 0:
            best = c
    assert best, n
    return best


_GC = 32
_JUNK = 64
_SG = 16
_NB = 4


def _pad_edges(src, dst, n_nodes):
    e = src.shape[0]
    unit = _GC * _NB * _NS * _NC
    ep = -(-e // unit) * unit
    pad = ep - e
    srcp = jnp.concatenate([src, jnp.zeros((pad,), src.dtype)])
    junk = n_nodes + (jnp.arange(pad, dtype=dst.dtype) % _JUNK)
    dstp = jnp.concatenate([dst, junk])
    return srcp, dstp


def _zero_init(z_hbm, acc, s, rows, tail):
    pltpu.sync_copy(z_hbm.at[pl.ds(s * rows, rows)], acc.at[pl.ds(s * rows, rows)])
    if tail:
        @pl.when(s == _NS - 1)
        def _():
            pltpu.sync_copy(z_hbm.at[pl.ds(_NS * rows, tail)],
                            acc.at[pl.ds(_NS * rows, tail)])


def _writeback(acc, out, s, rows, tail):
    pltpu.sync_copy(acc.at[pl.ds(s * rows, rows)], out.at[pl.ds(s * rows, rows)])
    if tail:
        @pl.when(s == _NS - 1)
        def _():
            pltpu.sync_copy(acc.at[pl.ds(_NS * rows, tail)],
                            out.at[pl.ds(_NS * rows, tail)])


def _sc_degree(dstp, n_nodes):
    ep = dstp.shape[0]
    ew = ep // (_NC * _NS)
    ng = ew // _SG
    assert ng * _SG * _NC * _NS == ep
    rows = (n_nodes // _NS) // 8 * 8
    tail = n_nodes - _NS * rows
    z = jnp.zeros((n_nodes, 128), jnp.float32)
    ones = jnp.ones((_SG, 128), jnp.float32)
    mesh = plsc.VectorSubcoreMesh(core_axis_name="c", subcore_axis_name="s")

    @functools.partial(
        pl.kernel,
        out_type=[jax.ShapeDtypeStruct((n_nodes, 128), jnp.float32)] * 2,
        mesh=mesh,
        scratch_types=[
            pltpu.VMEM((ew,), jnp.int32),
            pltpu.VMEM((_SG, 128), jnp.float32),
            pltpu.VMEM_SHARED((n_nodes + _JUNK, 128), jnp.float32),
        ],
    )
    def deg_kernel(dst_hbm, z_hbm, ones_hbm, out0, out1, dst_v, ones_v, acc):
        c = lax.axis_index("c")
        s = lax.axis_index("s")
        wid = s * _NC + c
        pltpu.sync_copy(ones_hbm, ones_v)
        pltpu.sync_copy(dst_hbm.at[pl.ds(wid * ew, ew)], dst_v)
        _zero_init(z_hbm, acc, s, rows, tail)
        plsc.subcore_barrier()

        def body(k, carry):
            idx = dst_v[pl.ds(k * _SG, _SG)]
            pltpu.sync_copy(ones_v, acc.at[idx], add=True)
            return carry

        lax.fori_loop(0, ng, body, 0)
        plsc.subcore_barrier()

        @pl.when(c == 0)
        def _():
            _writeback(acc, out0, s, rows, tail)

        @pl.when(c == 1)
        def _():
            _writeback(acc, out1, s, rows, tail)

    return deg_kernel(dstp, z, ones)


def _agg_pipeline(tab, acc, src_v, dst_v, rows_v, gsem, nchunks, nb):
    ngroups = nchunks // nb

    def issue(slot, k):
        pltpu.async_copy(tab.at[src_v.at[pl.ds(k * _GC, _GC)]],
                         rows_v[slot], gsem[slot])

    def drain(slot, k):
        pltpu.make_async_copy(tab.at[src_v.at[pl.ds(k * _GC, _GC)]],
                              rows_v[slot], gsem[slot]).wait()
        for j in range(_GC // _SG):
            idx = dst_v[pl.ds(k * _GC + j * _SG, _SG)]
            pltpu.sync_copy(rows_v[slot].at[pl.ds(j * _SG, _SG)],
                            acc.at[idx], add=True)

    for b in range(nb):
        issue(b, b)

    def group(g, carry):
        for b in range(nb):
            k = g * nb + b
            drain(b, k)
            issue(b, k + nb)
        return carry

    lax.fori_loop(0, ngroups - 1, group, 0)
    for b in range(nb):
        drain(b, (ngroups - 1) * nb + b)


def _sc_aggregate(tab_a, tab_b, srcp, dstp):
    n_nodes, dh = tab_a.shape
    ep = srcp.shape[0]
    ew = ep // _NS
    nch_s = ew // _GC
    assert nch_s * _GC * _NS == ep and nch_s % _NB == 0
    rows = (n_nodes // _NS) // 8 * 8
    tail = n_nodes - _NS * rows
    z = jnp.zeros((n_nodes, dh), jnp.float32)
    mesh = plsc.VectorSubcoreMesh(core_axis_name="c", subcore_axis_name="s")

    @functools.partial(
        pl.kernel,
        out_type=[jax.ShapeDtypeStruct((n_nodes, dh), jnp.float32)] * 2,
        mesh=mesh,
        scratch_types=[
            pltpu.VMEM((ew,), jnp.int32),
            pltpu.VMEM((ew,), jnp.int32),
            [pltpu.VMEM((_GC, dh), jnp.float32)] * _NB,
            [pltpu.SemaphoreType.DMA] * _NB,
            pltpu.VMEM_SHARED((n_nodes + _JUNK, dh), jnp.float32),
        ],
    )
    def agg_kernel(ta, tb, src_hbm, dst_hbm, z_hbm, out_a, out_b,
                   src_v, dst_v, rows_v, gsem, acc):
        c = lax.axis_index("c")
        s = lax.axis_index("s")

        def run(tab, out):
            pltpu.sync_copy(src_hbm.at[pl.ds(s * ew, ew)], src_v)
            pltpu.sync_copy(dst_hbm.at[pl.ds(s * ew, ew)], dst_v)
            _zero_init(z_hbm, acc, s, rows, tail)
            plsc.subcore_barrier()
            _agg_pipeline(tab, acc, src_v, dst_v, rows_v, gsem, nch_s, _NB)
            plsc.subcore_barrier()
            _writeback(acc, out, s, rows, tail)

        @pl.when(c == 0)
        def _():
            run(ta, out_a)

        @pl.when(c == 1)
        def _():
            run(tb, out_b)

    return agg_kernel(tab_a, tab_b, srcp, dstp, z)


def _sc_aggregate_edges(tab_a, tab_b, srcp, dstp):
    n_nodes, dh = tab_a.shape
    assert dh % 128 == 0
    ep = srcp.shape[0]
    ec = ep // _NC
    ew = ec // _NS
    nch_s = ew // _GC
    assert nch_s * _GC * _NS * _NC == ep and nch_s % _NB == 0
    rows = (n_nodes // _NS) // 8 * 8
    tail = n_nodes - _NS * rows
    z = jnp.zeros((n_nodes, dh), jnp.float32)
    mesh = plsc.VectorSubcoreMesh(core_axis_name="c", subcore_axis_name="s")

    @functools.partial(
        pl.kernel,
        out_type=[jax.ShapeDtypeStruct((n_nodes, dh), jnp.float32)] * 2,
        mesh=mesh,
        scratch_types=[
            pltpu.VMEM((ew,), jnp.int32),
            pltpu.VMEM((ew,), jnp.int32),
            [pltpu.VMEM((_GC, dh), jnp.float32)] * _NB,
            [pltpu.SemaphoreType.DMA] * _NB,
            pltpu.VMEM_SHARED((n_nodes + _JUNK, dh), jnp.float32),
        ],
    )
    def agg_kernel(ta, tb, src_hbm, dst_hbm, z_hbm, out0, out1,
                   src_v, dst_v, rows_v, gsem, acc):
        c = lax.axis_index("c")
        s = lax.axis_index("s")
        base = c * ec + s * ew
        pltpu.sync_copy(src_hbm.at[pl.ds(base, ew)], src_v)
        pltpu.sync_copy(dst_hbm.at[pl.ds(base, ew)], dst_v)
        _zero_init(z_hbm, acc, s, rows, tail)
        plsc.subcore_barrier()

        @pl.when(c == 0)
        def _():
            _agg_pipeline(ta, acc, src_v, dst_v, rows_v, gsem, nch_s, _NB)

        @pl.when(c == 1)
        def _():
            _agg_pipeline(tb, acc, src_v, dst_v, rows_v, gsem, nch_s, _NB)

        plsc.subcore_barrier()

        @pl.when(c == 0)
        def _():
            _writeback(acc, out0, s, rows, tail)

        @pl.when(c == 1)
        def _():
            _writeback(acc, out1, s, rows, tail)

    return agg_kernel(tab_a, tab_b, srcp, dstp, z)


def _dinv_from(d0, d1):
    deg = d0[:, :1] + d1[:, :1] + 1.0
    return lax.rsqrt(deg)


def _tc_pre(x, w1, d0, d1):
    n, f = x.shape
    hid = w1.shape[1]
    bn = 1000
    grid = n // bn

    def body(x_r, w_r, d0_r, d1_r, oa_r, ob_r):
        dinv = _dinv_from(d0_r[...], d1_r[...])
        r = jnp.dot(x_r[...], w_r[...], preferred_element_type=jnp.float32)
        hs = dinv * r
        oa_r[...] = hs[:, : hid // 2]
        ob_r[...] = hs[:, hid // 2:]

    return pl.pallas_call(
        body,
        grid=(grid,),
        in_specs=[
            pl.BlockSpec((bn, f), lambda i: (i, 0)),
            pl.BlockSpec((f, hid), lambda i: (0, 0)),
            pl.BlockSpec((bn, 128), lambda i: (i, 0)),
            pl.BlockSpec((bn, 128), lambda i: (i, 0)),
        ],
        out_specs=[
            pl.BlockSpec((bn, hid // 2), lambda i: (i, 0)),
            pl.BlockSpec((bn, hid // 2), lambda i: (i, 0)),
        ],
        out_shape=[jax.ShapeDtypeStruct((n, hid // 2), jnp.float32)] * 2,
    )(x, w1, d0, d1)


def _tc_mid(a1a, a1b, hs0a, hs0b, d0, d1, b1, wc):
    n = a1a.shape[0]
    hid = 2 * a1a.shape[1]
    dl2 = wc.shape[1]
    bn = 1000
    grid = n // bn

    def body(aa_r, ab_r, ha_r, hb_r, d0_r, d1_r, b1_r, w_r, oa_r, ob_r):
        dinv = _dinv_from(d0_r[...], d1_r[...])
        agg = jnp.concatenate([aa_r[...], ab_r[...]], axis=1)
        hs0 = jnp.concatenate([ha_r[...], hb_r[...]], axis=1)
        h = jnp.maximum(dinv * (agg + hs0) + b1_r[...], 0.0)
        hc = jnp.dot(h, w_r[...], preferred_element_type=jnp.float32)
        oa_r[...] = dinv * hc
        ob_r[...] = dinv * hc

    return pl.pallas_call(
        body,
        grid=(grid,),
        in_specs=[
            pl.BlockSpec((bn, hid // 2), lambda i: (i, 0)),
            pl.BlockSpec((bn, hid // 2), lambda i: (i, 0)),
            pl.BlockSpec((bn, hid // 2), lambda i: (i, 0)),
            pl.BlockSpec((bn, hid // 2), lambda i: (i, 0)),
            pl.BlockSpec((bn, 128), lambda i: (i, 0)),
            pl.BlockSpec((bn, 128), lambda i: (i, 0)),
            pl.BlockSpec((1, hid), lambda i: (0, 0)),
            pl.BlockSpec((hid, dl2), lambda i: (0, 0)),
        ],
        out_specs=[pl.BlockSpec((bn, dl2), lambda i: (i, 0))] * 2,
        out_shape=[jax.ShapeDtypeStruct((n, dl2), jnp.float32)] * 2,
    )(a1a, a1b, hs0a, hs0b, d0, d1, b1, wc)


def _tc_post(p0, p1, hsc, d0, d1, bmu, blv, eps):
    n, dl2 = p0.shape
    dl = dl2 // 2
    bn = 1000
    grid = n // bn

    def body(p0_r, p1_r, h_r, d0_r, d1_r, bmu_r, blv_r, eps_r, o_r):
        dinv = _dinv_from(d0_r[...], d1_r[...])
        g = dinv * (p0_r[...] + p1_r[...] + h_r[...])
        mu = g[:, :dl] + bmu_r[...]
        logvar = g[:, dl:] + blv_r[...]
        z = mu + eps_r[...] * jnp.exp(0.5 * logvar)
        o_r[...] = jnp.concatenate([z, mu, logvar], axis=1)

    return pl.pallas_call(
        body,
        grid=(grid,),
        in_specs=[
            pl.BlockSpec((bn, dl2), lambda i: (i, 0)),
            pl.BlockSpec((bn, dl2), lambda i: (i, 0)),
            pl.BlockSpec((bn, dl2), lambda i: (i, 0)),
            pl.BlockSpec((bn, 128), lambda i: (i, 0)),
            pl.BlockSpec((bn, 128), lambda i: (i, 0)),
            pl.BlockSpec((1, dl), lambda i: (0, 0)),
            pl.BlockSpec((1, dl), lambda i: (0, 0)),
            pl.BlockSpec((bn, dl), lambda i: (i, 0)),
        ],
        out_specs=pl.BlockSpec((bn, 3 * dl), lambda i: (i, 0)),
        out_shape=jax.ShapeDtypeStruct((n, 3 * dl), jnp.float32),
    )(p0, p1, hsc, d0, d1, bmu, blv, eps)


def kernel(x, edge_index, W1, b1, Wmu, bmu, Wlv, blv, eps):
    n = x.shape[0]
    src2d, dst2d = _pad_edges(edge_index[0], edge_index[1], n)
    d0, d1 = _sc_degree(dst2d, n)
    hs0a, hs0b = _tc_pre(x, W1, d0, d1)
    a1a, a1b = _sc_aggregate(hs0a, hs0b, src2d, dst2d)
    wc = jnp.concatenate([Wmu, Wlv], axis=1)
    hsca, hscb = _tc_mid(a1a, a1b, hs0a, hs0b, d0, d1, b1.reshape(1, -1), wc)
    p0, p1 = _sc_aggregate_edges(hsca, hscb, src2d, dst2d)
    return _tc_post(p0, p1, hsca, d0, d1,
                    bmu.reshape(1, -1), blv.reshape(1, -1), eps)


# speedup vs baseline: 2.2351x; 2.2351x over previous
import functools

import jax
import jax.numpy as jnp
from jax import lax
from jax.experimental import pallas as pl
from jax.experimental.pallas import tpu as pltpu
from jax.experimental.pallas import tpu_sc as plsc

_NC, _NS = 2, 16


def _chunk_size(n, cap=128):
    best = 0
    for c in range(8, cap + 1, 8):
        if n % c == 0:
            best = c
    assert best, n
    return best


_GC = 32
_JUNK = 64
_SG = 16
_NB = 4


def _pad_edges(src, dst, n_nodes):
    e = src.shape[0]
    unit = _GC * _NB * _NS * _NC
    ep = -(-e // unit) * unit
    pad = ep - e
    fill = jnp.arange(pad, dtype=src.dtype)
    srcp = jnp.concatenate([src, fill % jnp.int32(n_nodes)])
    dstp = jnp.concatenate([dst, n_nodes + fill % _JUNK])
    return srcp, dstp


def _zero_init(z_hbm, acc, s, rows, tail):
    pltpu.sync_copy(z_hbm.at[pl.ds(s * rows, rows)], acc.at[pl.ds(s * rows, rows)])
    if tail:
        @pl.when(s == _NS - 1)
        def _():
            pltpu.sync_copy(z_hbm.at[pl.ds(_NS * rows, tail)],
                            acc.at[pl.ds(_NS * rows, tail)])


def _writeback(acc, out, s, rows, tail):
    pltpu.sync_copy(acc.at[pl.ds(s * rows, rows)], out.at[pl.ds(s * rows, rows)])
    if tail:
        @pl.when(s == _NS - 1)
        def _():
            pltpu.sync_copy(acc.at[pl.ds(_NS * rows, tail)],
                            out.at[pl.ds(_NS * rows, tail)])


def _sc_degree(dstp, n_nodes):
    ep = dstp.shape[0]
    ew = ep // (_NC * _NS)
    ng = ew // _SG
    assert ng * _SG * _NC * _NS == ep
    rows = (n_nodes // _NS) // 8 * 8
    tail = n_nodes - _NS * rows
    z = jnp.zeros((n_nodes, 128), jnp.float32)
    ones = jnp.ones((_SG, 128), jnp.float32)
    mesh = plsc.VectorSubcoreMesh(core_axis_name="c", subcore_axis_name="s")

    @functools.partial(
        pl.kernel,
        out_type=[jax.ShapeDtypeStruct((n_nodes, 128), jnp.float32)] * 2,
        mesh=mesh,
        scratch_types=[
            pltpu.VMEM((ew,), jnp.int32),
            pltpu.VMEM((_SG, 128), jnp.float32),
            pltpu.VMEM_SHARED((n_nodes + _JUNK, 128), jnp.float32),
        ],
    )
    def deg_kernel(dst_hbm, z_hbm, ones_hbm, out0, out1, dst_v, ones_v, acc):
        c = lax.axis_index("c")
        s = lax.axis_index("s")
        wid = s * _NC + c
        pltpu.sync_copy(ones_hbm, ones_v)
        pltpu.sync_copy(dst_hbm.at[pl.ds(wid * ew, ew)], dst_v)
        _zero_init(z_hbm, acc, s, rows, tail)
        plsc.subcore_barrier()

        def body(k, carry):
            idx = dst_v[pl.ds(k * _SG, _SG)]
            pltpu.sync_copy(ones_v, acc.at[idx], add=True)
            return carry

        lax.fori_loop(0, ng, body, 0)
        plsc.subcore_barrier()

        @pl.when(c == 0)
        def _():
            _writeback(acc, out0, s, rows, tail)

        @pl.when(c == 1)
        def _():
            _writeback(acc, out1, s, rows, tail)

    return deg_kernel(dstp, z, ones)


def _agg_pipeline(tab, acc, src_v, dst_v, rows_v, gsem, nchunks, nb):
    ngroups = nchunks // nb

    def issue(slot, k):
        pltpu.async_copy(tab.at[src_v.at[pl.ds(k * _GC, _GC)]],
                         rows_v[slot], gsem[slot])

    def drain(slot, k):
        pltpu.make_async_copy(tab.at[src_v.at[pl.ds(k * _GC, _GC)]],
                              rows_v[slot], gsem[slot]).wait()
        for j in range(_GC // _SG):
            idx = dst_v[pl.ds(k * _GC + j * _SG, _SG)]
            pltpu.sync_copy(rows_v[slot].at[pl.ds(j * _SG, _SG)],
                            acc.at[idx], add=True)

    for b in range(nb):
        issue(b, b)

    def group(g, carry):
        for b in range(nb):
            k = g * nb + b
            drain(b, k)
            issue(b, k + nb)
        return carry

    lax.fori_loop(0, ngroups - 1, group, 0)
    for b in range(nb):
        drain(b, (ngroups - 1) * nb + b)


def _sc_aggregate(tab_a, tab_b, srcp, dstp):
    n_nodes, dh = tab_a.shape
    ep = srcp.shape[0]
    ew = ep // _NS
    nch_s = ew // _GC
    assert nch_s * _GC * _NS == ep and nch_s % _NB == 0
    rows = (n_nodes // _NS) // 8 * 8
    tail = n_nodes - _NS * rows
    z = jnp.zeros((n_nodes, dh), jnp.float32)
    mesh = plsc.VectorSubcoreMesh(core_axis_name="c", subcore_axis_name="s")

    @functools.partial(
        pl.kernel,
        out_type=[jax.ShapeDtypeStruct((n_nodes, dh), jnp.float32)] * 2,
        mesh=mesh,
        scratch_types=[
            pltpu.VMEM((ew,), jnp.int32),
            pltpu.VMEM((ew,), jnp.int32),
            [pltpu.VMEM((_GC, dh), jnp.float32)] * _NB,
            [pltpu.SemaphoreType.DMA] * _NB,
            pltpu.VMEM_SHARED((n_nodes + _JUNK, dh), jnp.float32),
        ],
    )
    def agg_kernel(ta, tb, src_hbm, dst_hbm, z_hbm, out_a, out_b,
                   src_v, dst_v, rows_v, gsem, acc):
        c = lax.axis_index("c")
        s = lax.axis_index("s")

        def run(tab, out):
            pltpu.sync_copy(src_hbm.at[pl.ds(s * ew, ew)], src_v)
            pltpu.sync_copy(dst_hbm.at[pl.ds(s * ew, ew)], dst_v)
            _zero_init(z_hbm, acc, s, rows, tail)
            plsc.subcore_barrier()
            _agg_pipeline(tab, acc, src_v, dst_v, rows_v, gsem, nch_s, _NB)
            plsc.subcore_barrier()
            _writeback(acc, out, s, rows, tail)

        @pl.when(c == 0)
        def _():
            run(ta, out_a)

        @pl.when(c == 1)
        def _():
            run(tb, out_b)

    return agg_kernel(tab_a, tab_b, srcp, dstp, z)


def _sc_aggregate_edges(tab, srcp, dstp):
    n_nodes, dh = tab.shape
    assert dh % 128 == 0
    ep = srcp.shape[0]
    ec = ep // _NC
    ew = ec // _NS
    nch_s = ew // _GC
    assert nch_s * _GC * _NS * _NC == ep and nch_s % _NB == 0
    rows = (n_nodes // _NS) // 8 * 8
    tail = n_nodes - _NS * rows
    z = jnp.zeros((n_nodes, dh), jnp.float32)
    mesh = plsc.VectorSubcoreMesh(core_axis_name="c", subcore_axis_name="s")

    @functools.partial(
        pl.kernel,
        out_type=[jax.ShapeDtypeStruct((n_nodes, dh), jnp.float32)] * 2,
        mesh=mesh,
        scratch_types=[
            pltpu.VMEM((ew,), jnp.int32),
            pltpu.VMEM((ew,), jnp.int32),
            [pltpu.VMEM((_GC, dh), jnp.float32)] * _NB,
            [pltpu.SemaphoreType.DMA] * _NB,
            pltpu.VMEM_SHARED((n_nodes + _JUNK, dh), jnp.float32),
        ],
    )
    def agg_kernel(tab_hbm, src_hbm, dst_hbm, z_hbm, out0, out1,
                   src_v, dst_v, rows_v, gsem, acc):
        c = lax.axis_index("c")
        s = lax.axis_index("s")
        base = c * ec + s * ew
        pltpu.sync_copy(src_hbm.at[pl.ds(base, ew)], src_v)
        pltpu.sync_copy(dst_hbm.at[pl.ds(base, ew)], dst_v)
        _zero_init(z_hbm, acc, s, rows, tail)
        plsc.subcore_barrier()
        _agg_pipeline(tab_hbm, acc, src_v, dst_v, rows_v, gsem, nch_s, _NB)
        plsc.subcore_barrier()

        @pl.when(c == 0)
        def _():
            _writeback(acc, out0, s, rows, tail)

        @pl.when(c == 1)
        def _():
            _writeback(acc, out1, s, rows, tail)

    return agg_kernel(tab, srcp, dstp, z)


def _dinv_from(d0, d1):
    deg = d0[:, :1] + d1[:, :1] + 1.0
    return lax.rsqrt(deg)


def _tc_pre(x, w1, d0, d1):
    n, f = x.shape
    hid = w1.shape[1]
    bn = 1000
    grid = n // bn

    def body(x_r, w_r, d0_r, d1_r, oa_r, ob_r):
        dinv = _dinv_from(d0_r[...], d1_r[...])
        r = jnp.dot(x_r[...], w_r[...], preferred_element_type=jnp.float32)
        hs = dinv * r
        oa_r[...] = hs[:, : hid // 2]
        ob_r[...] = hs[:, hid // 2:]

    return pl.pallas_call(
        body,
        grid=(grid,),
        in_specs=[
            pl.BlockSpec((bn, f), lambda i: (i, 0)),
            pl.BlockSpec((f, hid), lambda i: (0, 0)),
            pl.BlockSpec((bn, 128), lambda i: (i, 0)),
            pl.BlockSpec((bn, 128), lambda i: (i, 0)),
        ],
        out_specs=[
            pl.BlockSpec((bn, hid // 2), lambda i: (i, 0)),
            pl.BlockSpec((bn, hid // 2), lambda i: (i, 0)),
        ],
        out_shape=[jax.ShapeDtypeStruct((n, hid // 2), jnp.float32)] * 2,
    )(x, w1, d0, d1)


def _tc_mid(a1a, a1b, hs0a, hs0b, d0, d1, b1, wc):
    n = a1a.shape[0]
    hid = 2 * a1a.shape[1]
    dl2 = wc.shape[1]
    bn = 1000
    grid = n // bn

    def body(aa_r, ab_r, ha_r, hb_r, d0_r, d1_r, b1_r, w_r, o_r):
        dinv = _dinv_from(d0_r[...], d1_r[...])
        agg = jnp.concatenate([aa_r[...], ab_r[...]], axis=1)
        hs0 = jnp.concatenate([ha_r[...], hb_r[...]], axis=1)
        h = jnp.maximum(dinv * (agg + hs0) + b1_r[...], 0.0)
        hc = jnp.dot(h, w_r[...], preferred_element_type=jnp.float32)
        o_r[...] = dinv * hc

    return pl.pallas_call(
        body,
        grid=(grid,),
        in_specs=[
            pl.BlockSpec((bn, hid // 2), lambda i: (i, 0)),
            pl.BlockSpec((bn, hid // 2), lambda i: (i, 0)),
            pl.BlockSpec((bn, hid // 2), lambda i: (i, 0)),
            pl.BlockSpec((bn, hid // 2), lambda i: (i, 0)),
            pl.BlockSpec((bn, 128), lambda i: (i, 0)),
            pl.BlockSpec((bn, 128), lambda i: (i, 0)),
            pl.BlockSpec((1, hid), lambda i: (0, 0)),
            pl.BlockSpec((hid, dl2), lambda i: (0, 0)),
        ],
        out_specs=pl.BlockSpec((bn, dl2), lambda i: (i, 0)),
        out_shape=jax.ShapeDtypeStruct((n, dl2), jnp.float32),
    )(a1a, a1b, hs0a, hs0b, d0, d1, b1, wc)


def _tc_post(p0, p1, hsc, d0, d1, bmu, blv, eps):
    n, dl2 = p0.shape
    dl = dl2 // 2
    bn = 1000
    grid = n // bn

    def body(p0_r, p1_r, h_r, d0_r, d1_r, bmu_r, blv_r, eps_r, o_r):
        dinv = _dinv_from(d0_r[...], d1_r[...])
        g = dinv * (p0_r[...] + p1_r[...] + h_r[...])
        mu = g[:, :dl] + bmu_r[...]
        logvar = g[:, dl:] + blv_r[...]
        z = mu + eps_r[...] * jnp.exp(0.5 * logvar)
        o_r[...] = jnp.concatenate([z, mu, logvar], axis=1)

    return pl.pallas_call(
        body,
        grid=(grid,),
        in_specs=[
            pl.BlockSpec((bn, dl2), lambda i: (i, 0)),
            pl.BlockSpec((bn, dl2), lambda i: (i, 0)),
            pl.BlockSpec((bn, dl2), lambda i: (i, 0)),
            pl.BlockSpec((bn, 128), lambda i: (i, 0)),
            pl.BlockSpec((bn, 128), lambda i: (i, 0)),
            pl.BlockSpec((1, dl), lambda i: (0, 0)),
            pl.BlockSpec((1, dl), lambda i: (0, 0)),
            pl.BlockSpec((bn, dl), lambda i: (i, 0)),
        ],
        out_specs=pl.BlockSpec((bn, 3 * dl), lambda i: (i, 0)),
        out_shape=jax.ShapeDtypeStruct((n, 3 * dl), jnp.float32),
    )(p0, p1, hsc, d0, d1, bmu, blv, eps)


def kernel(x, edge_index, W1, b1, Wmu, bmu, Wlv, blv, eps):
    n = x.shape[0]
    src2d, dst2d = _pad_edges(edge_index[0], edge_index[1], n)
    d0, d1 = _sc_degree(dst2d, n)
    hs0a, hs0b = _tc_pre(x, W1, d0, d1)
    a1a, a1b = _sc_aggregate(hs0a, hs0b, src2d, dst2d)
    wc = jnp.concatenate([Wmu, Wlv], axis=1)
    hsc = _tc_mid(a1a, a1b, hs0a, hs0b, d0, d1, b1.reshape(1, -1), wc)
    p0, p1 = _sc_aggregate_edges(hsc, src2d, dst2d)
    return _tc_post(p0, p1, hsc, d0, d1,
                    bmu.reshape(1, -1), blv.reshape(1, -1), eps)
